# baseline (speedup 1.0000x reference)
import jax
import jax.numpy as jnp
from jax.experimental import pallas as pl


def _body(vel_ref, pos_ref, hed_ref, speed_ref, ef_ref,
          Wh_ref, bh_ref, Wp_ref, bp_ref, Wd_ref, bd_ref, We_ref, be_ref,
          h_ref, p_ref, d_ref, e_ref):
    vel = vel_ref[...]
    pos = pos_ref[...]
    hed = hed_ref[...]
    sp = speed_ref[...]
    Wh = Wh_ref[...]
    Wp = Wp_ref[...]
    Wd = Wd_ref[...]

    h = bh_ref[...]
    h = h + vel[:, 0:1] * Wh[0:1, :] + vel[:, 1:2] * Wh[1:2, :]
    h = h + pos[:, 0:1] * Wh[2:3, :] + pos[:, 1:2] * Wh[3:4, :]
    h = h + hed[:, 0:1] * Wh[4:5, :] + hed[:, 1:2] * Wh[5:6, :]
    h_ref[...] = h + sp * Wh[6:7, :]

    p_ref[...] = (bp_ref[...]
                  + pos[:, 0:1] * Wp[0:1, :]
                  + pos[:, 1:2] * Wp[1:2, :])

    rh = jnp.round(hed)
    d_ref[...] = (bd_ref[...]
                  + (rh[:, 0:1] * sp) * Wd[0:1, :]
                  + (rh[:, 1:2] * sp) * Wd[1:2, :])

    ef = ef_ref[...]
    We = We_ref[...]
    e = be_ref[...]
    e = e + ef[:, 0:1] * We[0:1, :] + ef[:, 1:2] * We[1:2, :]
    e_ref[...] = e + ef[:, 2:3] * We[2:3, :] + ef[:, 3:4] * We[3:4, :]


def kernel(vel, pos, hed, speed, e_feat, edge_index,
           W_h, b_h, W_p, b_p, W_d, b_d, W_e, b_e):
    del edge_index
    n = vel.shape[0]
    e_rows = e_feat.shape[0]
    hdim = W_h.shape[1]

    grid = 250
    bn = n // grid
    be = e_rows // grid

    b_h2 = b_h.reshape(1, hdim)
    b_p2 = b_p.reshape(1, hdim)
    b_d2 = b_d.reshape(1, hdim)
    b_e2 = b_e.reshape(1, hdim)

    def nmap(i):
        return (i, 0)

    def wmap(i):
        return (0, 0)

    out = pl.pallas_call(
        _body,
        grid=(grid,),
        in_specs=[
            pl.BlockSpec((bn, 2), nmap),
            pl.BlockSpec((bn, 2), nmap),
            pl.BlockSpec((bn, 2), nmap),
            pl.BlockSpec((bn, 1), nmap),
            pl.BlockSpec((be, 4), nmap),
            pl.BlockSpec((7, hdim), wmap),
            pl.BlockSpec((1, hdim), wmap),
            pl.BlockSpec((2, hdim), wmap),
            pl.BlockSpec((1, hdim), wmap),
            pl.BlockSpec((2, hdim), wmap),
            pl.BlockSpec((1, hdim), wmap),
            pl.BlockSpec((4, hdim), wmap),
            pl.BlockSpec((1, hdim), wmap),
        ],
        out_specs=[
            pl.BlockSpec((bn, hdim), nmap),
            pl.BlockSpec((bn, hdim), nmap),
            pl.BlockSpec((bn, hdim), nmap),
            pl.BlockSpec((be, hdim), nmap),
        ],
        out_shape=[
            jax.ShapeDtypeStruct((n, hdim), jnp.float32),
            jax.ShapeDtypeStruct((n, hdim), jnp.float32),
            jax.ShapeDtypeStruct((n, hdim), jnp.float32),
            jax.ShapeDtypeStruct((e_rows, hdim), jnp.float32),
        ],
    )(vel, pos, hed, speed, e_feat,
      W_h, b_h2, W_p, b_p2, W_d, b_d2, W_e, b_e2)

    return tuple(out)


# speedup vs baseline: 1.3608x; 1.3608x over previous
import jax
import jax.numpy as jnp
from jax.experimental import pallas as pl


def _body(nf_ref, ef_ref, Whpd_ref, Wd_ref, bhpd_ref, We_ref, be_ref,
          h_ref, p_ref, d_ref, e_ref):
    nf = nf_ref[...]
    rs = jnp.round(nf[:, 4:6]) * nf[:, 6:7]
    hpd = (jnp.dot(nf, Whpd_ref[...], preferred_element_type=jnp.float32)
           + jnp.dot(rs, Wd_ref[...], preferred_element_type=jnp.float32)
           + bhpd_ref[...])
    h_ref[...] = hpd[:, 0:64]
    p_ref[...] = hpd[:, 64:128]
    d_ref[...] = hpd[:, 128:192]

    e_ref[...] = (jnp.dot(ef_ref[...], We_ref[...],
                          preferred_element_type=jnp.float32)
                  + be_ref[...])


def kernel(vel, pos, hed, speed, e_feat, edge_index,
           W_h, b_h, W_p, b_p, W_d, b_d, W_e, b_e):
    del edge_index
    n = vel.shape[0]
    e_rows = e_feat.shape[0]
    hdim = W_h.shape[1]
    f32 = jnp.float32

    nf = jnp.concatenate(
        [vel, pos, hed, speed, jnp.zeros((n, 1), f32)], axis=1)

    z = jnp.zeros((1, hdim), f32)
    w_h8 = jnp.concatenate([W_h, z], axis=0)
    w_p8 = jnp.concatenate(
        [jnp.zeros((2, hdim), f32), W_p, jnp.zeros((4, hdim), f32)], axis=0)
    w_hpd = jnp.concatenate(
        [w_h8, w_p8, jnp.zeros((8, hdim), f32)], axis=1)
    w_d3 = jnp.concatenate(
        [jnp.zeros((2, 2 * hdim), f32), W_d], axis=1)
    b_hpd = jnp.concatenate([b_h, b_p, b_d]).reshape(1, 3 * hdim)
    b_e2 = b_e.reshape(1, hdim)

    grid = 50
    bn = n // grid
    be = e_rows // grid

    def rmap(i):
        return (i, 0)

    def wmap(i):
        return (0, 0)

    out = pl.pallas_call(
        _body,
        grid=(grid,),
        in_specs=[
            pl.BlockSpec((bn, 8), rmap),
            pl.BlockSpec((be, 4), rmap),
            pl.BlockSpec((8, 3 * hdim), wmap),
            pl.BlockSpec((2, 3 * hdim), wmap),
            pl.BlockSpec((1, 3 * hdim), wmap),
            pl.BlockSpec((4, hdim), wmap),
            pl.BlockSpec((1, hdim), wmap),
        ],
        out_specs=[
            pl.BlockSpec((bn, hdim), rmap),
            pl.BlockSpec((bn, hdim), rmap),
            pl.BlockSpec((bn, hdim), rmap),
            pl.BlockSpec((be, hdim), rmap),
        ],
        out_shape=[
            jax.ShapeDtypeStruct((n, hdim), f32),
            jax.ShapeDtypeStruct((n, hdim), f32),
            jax.ShapeDtypeStruct((n, hdim), f32),
            jax.ShapeDtypeStruct((e_rows, hdim), f32),
        ],
    )(nf, e_feat, w_hpd, w_d3, b_hpd, W_e, b_e2)

    return tuple(out)


# speedup vs baseline: 12.0164x; 8.8307x over previous
import jax
import jax.numpy as jnp
from jax.experimental import pallas as pl


def _body(nf_ref, ef_ref, Whpd_ref, Wd_ref, bhpd_ref, We_ref, be_ref,
          h_ref, p_ref, d_ref, e_ref):
    nf = nf_ref[...]
    rs = jnp.round(nf[4:6, :]) * nf[6:7, :]
    hpd = (jnp.dot(Whpd_ref[...], nf, preferred_element_type=jnp.float32)
           + jnp.dot(Wd_ref[...], rs, preferred_element_type=jnp.float32)
           + bhpd_ref[...][:, 0:1])
    h_ref[...] = hpd[0:64, :]
    p_ref[...] = hpd[64:128, :]
    d_ref[...] = hpd[128:192, :]

    e_ref[...] = (jnp.dot(We_ref[...], ef_ref[...],
                          preferred_element_type=jnp.float32)
                  + be_ref[...][:, 0:1])


def kernel(vel, pos, hed, speed, e_feat, edge_index,
           W_h, b_h, W_p, b_p, W_d, b_d, W_e, b_e):
    del edge_index
    n = vel.shape[0]
    e_rows = e_feat.shape[0]
    hdim = W_h.shape[1]
    f32 = jnp.float32

    nf = jnp.concatenate(
        [vel.T, pos.T, hed.T, speed.T, jnp.zeros((1, n), f32)], axis=0)
    ef = e_feat.T

    z = jnp.zeros((hdim, 1), f32)
    w_h8 = jnp.concatenate([W_h.T, z], axis=1)
    w_p8 = jnp.concatenate(
        [jnp.zeros((hdim, 2), f32), W_p.T, jnp.zeros((hdim, 4), f32)], axis=1)
    w_hpd = jnp.concatenate(
        [w_h8, w_p8, jnp.zeros((hdim, 8), f32)], axis=0)
    w_d3 = jnp.concatenate(
        [jnp.zeros((2 * hdim, 2), f32), W_d.T], axis=0)
    b_hpd = jnp.concatenate([b_h, b_p, b_d]).reshape(3 * hdim, 1)
    b_e2 = b_e.reshape(hdim, 1)

    grid = 25
    bn = 2048
    be = e_rows // grid

    def cmap(i):
        return (0, i)

    def wmap(i):
        return (0, 0)

    out = pl.pallas_call(
        _body,
        grid=(grid,),
        in_specs=[
            pl.BlockSpec((8, bn), cmap),
            pl.BlockSpec((4, be), cmap),
            pl.BlockSpec((3 * hdim, 8), wmap),
            pl.BlockSpec((3 * hdim, 2), wmap),
            pl.BlockSpec((3 * hdim, 1), wmap),
            pl.BlockSpec((hdim, 4), wmap),
            pl.BlockSpec((hdim, 1), wmap),
        ],
        out_specs=[
            pl.BlockSpec((hdim, bn), cmap),
            pl.BlockSpec((hdim, bn), cmap),
            pl.BlockSpec((hdim, bn), cmap),
            pl.BlockSpec((hdim, be), cmap),
        ],
        out_shape=[
            jax.ShapeDtypeStruct((hdim, n), f32),
            jax.ShapeDtypeStruct((hdim, n), f32),
            jax.ShapeDtypeStruct((hdim, n), f32),
            jax.ShapeDtypeStruct((hdim, e_rows), f32),
        ],
    )(nf, ef, w_hpd, w_d3, b_hpd, W_e.T, b_e2)

    return (out[0].T, out[1].T, out[2].T, out[3].T)


# speedup vs baseline: 13.4384x; 1.1183x over previous
import jax
import jax.numpy as jnp
from jax.experimental import pallas as pl


def _dg0(w, x):
    return jax.lax.dot_general(w, x, (((0,), (0,)), ((), ())),
                               preferred_element_type=jnp.float32)


def _body(vel_ref, pos_ref, hed_ref, speed_ref, ef_ref, pk_ref, bcol_ref,
          h_ref, p_ref, d_ref, e_ref):
    vel = vel_ref[...]
    pos = pos_ref[...]
    hed = hed_ref[...]
    sp = speed_ref[...]
    ones = jnp.ones(sp.shape, jnp.float32)
    nf8 = jnp.concatenate([vel, pos, hed, sp, ones], axis=0)
    pk = pk_ref[...]

    h_ref[...] = _dg0(pk[:, 0:64], nf8)
    p_ref[...] = _dg0(pk[:, 64:128], nf8)

    rs3 = jnp.concatenate([jnp.round(hed) * sp, ones], axis=0)
    d_ref[...] = _dg0(pk[0:3, 128:192], rs3)

    e_ref[...] = _dg0(pk[0:4, 192:256], ef_ref[...]) + bcol_ref[...][:, 0:1]


def kernel(vel, pos, hed, speed, e_feat, edge_index,
           W_h, b_h, W_p, b_p, W_d, b_d, W_e, b_e):
    del edge_index
    n = vel.shape[0]
    e_rows = e_feat.shape[0]
    hdim = W_h.shape[1]
    f32 = jnp.float32

    z1 = jnp.zeros((1, hdim), f32)
    z2 = jnp.zeros((2, hdim), f32)
    c_h = jnp.concatenate([W_h, b_h[None, :]], axis=0)
    c_p = jnp.concatenate([z2, W_p, z2, z1, b_p[None, :]], axis=0)
    c_d = jnp.concatenate([W_d, b_d[None, :], z2, z2, z1], axis=0)
    c_e = jnp.concatenate([W_e, z2, z2], axis=0)
    pk = jnp.concatenate([c_h, c_p, c_d, c_e], axis=1)
    bcol = b_e.reshape(hdim, 1)

    grid = 25
    bn = 2048
    be = e_rows // grid

    def cmap(i):
        return (0, i)

    def wmap(i):
        return (0, 0)

    out = pl.pallas_call(
        _body,
        grid=(grid,),
        in_specs=[
            pl.BlockSpec((2, bn), cmap),
            pl.BlockSpec((2, bn), cmap),
            pl.BlockSpec((2, bn), cmap),
            pl.BlockSpec((1, bn), cmap),
            pl.BlockSpec((4, be), cmap),
            pl.BlockSpec((8, 4 * hdim), wmap),
            pl.BlockSpec((hdim, 1), wmap),
        ],
        out_specs=[
            pl.BlockSpec((hdim, bn), cmap),
            pl.BlockSpec((hdim, bn), cmap),
            pl.BlockSpec((hdim, bn), cmap),
            pl.BlockSpec((hdim, be), cmap),
        ],
        out_shape=[
            jax.ShapeDtypeStruct((hdim, n), f32),
            jax.ShapeDtypeStruct((hdim, n), f32),
            jax.ShapeDtypeStruct((hdim, n), f32),
            jax.ShapeDtypeStruct((hdim, e_rows), f32),
        ],
    )(vel.T, pos.T, hed.T, speed.T, e_feat.T, pk, bcol)

    return (out[0].T, out[1].T, out[2].T, out[3].T)
